# baseline (speedup 1.0000x reference)
import functools

import jax
import jax.numpy as jnp
from jax.experimental import pallas as pl
from jax.experimental.pallas import tpu as pltpu

BB = 64
SS = 72


def _spinn_kernel(code_ref, bias_g_ref, bias_c_ref, th0_ref, tc0_ref,
                  seq_hbm, w_hbm, u_hbm,
                  out_ref,
                  seq_v, w_v, u_v, top_b, sec_b, buf_b, deep_b, mval_b,
                  stack_v, sem_seq, sem_w, sem_u,
                  *, T, L, H, TD):
    i = pl.program_id(0)
    nrows = BB * L
    cp_seq = pltpu.make_async_copy(seq_hbm.at[pl.ds(i * nrows, nrows), :],
                                   seq_v, sem_seq)
    cp_w = pltpu.make_async_copy(w_hbm, w_v, sem_w)
    cp_u = pltpu.make_async_copy(u_hbm, u_v, sem_u)
    cp_seq.start()
    cp_w.start()
    cp_u.start()
    cp_seq.wait()

    for mi in range(BB):
        t0 = seq_v[pl.ds(mi * L, 1), :]
        top_b[pl.ds(mi, 1), :] = t0
        sec_b[pl.ds(mi, 1), :] = t0

    cp_w.wait()
    cp_u.wait()

    def step(t, carry):
        th, tc = carry

        iota8g = jax.lax.broadcasted_iota(jnp.int32, (8, 1), 0)
        for mi in range(BB):
            c = code_ref[mi, t]
            d = c & 127
            bp = (c >> 7) & 127
            trs = (c >> 21) & 3
            bbase = pl.multiple_of(mi * L + ((bp >> 3) << 3), 8)
            bchunk = seq_v[pl.ds(bbase, 8), :]
            buf_b[pl.ds(mi, 1), :] = pltpu.roll(bchunk, -(bp & 7), axis=0)[0:1, :]
            mval_b[pl.ds(mi, 1), :] = jnp.full((1, 128), trs, jnp.float32)

            pbase = pl.multiple_of(mi * SS + (((d - 2) >> 3) << 3), 8)
            rbase = pl.multiple_of(mi * SS + (((d - 3) >> 3) << 3), 8)

            @pl.when(trs == 1)
            def _push():
                ch = stack_v[pl.ds(pbase, 8), :]
                sv = jnp.broadcast_to(sec_b[pl.ds(mi, 1), :], (8, 2 * H))
                stack_v[pl.ds(pbase, 8), :] = jnp.where(
                    iota8g == ((d - 2) & 7), sv, ch)

            @pl.when(trs != 1)
            def _pop():
                ch = stack_v[pl.ds(rbase, 8), :]
                deep_b[pl.ds(mi, 1), :] = pltpu.roll(
                    ch, -((d - 3) & 7), axis=0)[0:1, :]

        tcol = mval_b[:, 0:1]
        lm = tcol == 2.0
        rm = tcol == 3.0
        sm = tcol == 1.0

        top = top_b[...]
        sec = sec_b[...]
        buf = buf_b[...]
        top_h, top_c = top[:, :H], top[:, H:]
        sec_h, sec_c = sec[:, :H], sec[:, H:]
        buf_h, buf_c = buf[:, :H], buf[:, H:]

        gates = (jnp.dot(buf_h, w_v[0:H, :], preferred_element_type=jnp.float32)
                 + jnp.dot(top_h, w_v[H:2 * H, :], preferred_element_type=jnp.float32)
                 + jnp.dot(sec_h, w_v[2 * H:3 * H, :], preferred_element_type=jnp.float32)
                 + jnp.dot(th, w_v[3 * H:3 * H + TD, :], preferred_element_type=jnp.float32)
                 + bias_g_ref[...])
        ig = gates[:, 0:TD]
        fg = gates[:, TD:2 * TD]
        gg = gates[:, 2 * TD:3 * TD]
        og = gates[:, 3 * TD:4 * TD]
        tc_n = jax.nn.sigmoid(fg) * tc + jax.nn.sigmoid(ig) * jnp.tanh(gg)
        th_n = jax.nn.sigmoid(og) * jnp.tanh(tc_n)

        head_h = jnp.where(lm, top_h, sec_h)
        head_c = jnp.where(lm, top_c, sec_c)
        child_h = jnp.where(lm, sec_h, top_h)
        child_c = jnp.where(lm, sec_c, top_c)
        zeros = jnp.zeros_like(child_h)
        cl_in = jnp.where(lm, child_h, zeros)
        cr_in = jnp.where(rm, child_h, zeros)
        cg = (jnp.dot(head_h, u_v[0:H, :], preferred_element_type=jnp.float32)
              + jnp.dot(cl_in, u_v[H:2 * H, :], preferred_element_type=jnp.float32)
              + jnp.dot(cr_in, u_v[2 * H:3 * H, :], preferred_element_type=jnp.float32)
              + jnp.dot(th_n, u_v[3 * H:3 * H + TD, :], preferred_element_type=jnp.float32)
              + bias_c_ref[...])
        i_j = cg[:, 0:H]
        o_j = cg[:, H:2 * H]
        f_jh = cg[:, 2 * H:3 * H]
        f_jc = cg[:, 3 * H:4 * H]
        u_j = cg[:, 4 * H:5 * H]
        c_new = (jax.nn.sigmoid(i_j) * jnp.tanh(u_j)
                 + jax.nn.sigmoid(f_jh) * head_c
                 + jax.nn.sigmoid(f_jc) * child_c)
        h_new = jax.nn.sigmoid(o_j) * jnp.tanh(c_new)

        comp = jnp.concatenate([h_new, c_new], axis=1)
        top_b[...] = jnp.where(sm, buf, comp)
        sec_b[...] = jnp.where(sm, top, deep_b[...])

        return th_n, tc_n

    jax.lax.fori_loop(0, T, step, (th0_ref[...], tc0_ref[...]))

    out_ref[...] = top_b[:, 0:H]


@jax.jit
def kernel(sequence, transitions, W_comp, U_head_w, U_head_b, U_cl, U_cr,
           W_ih, W_hh, b_ih, b_hh, th0, tc0):
    B, L, H2 = sequence.shape
    H = H2 // 2
    TD = W_hh.shape[0]
    T = transitions.shape[1]

    tr = transitions.astype(jnp.int32)
    is_sh = (tr == 1).astype(jnp.int32)
    sp_after = 2 + jnp.cumsum(2 * is_sh - 1, axis=1)
    sp_before = jnp.concatenate(
        [jnp.full((B, 1), 2, jnp.int32), sp_after[:, :-1]], axis=1)
    shifts_before = jnp.concatenate(
        [jnp.zeros((B, 1), jnp.int32), jnp.cumsum(is_sh, axis=1)[:, :-1]], axis=1)
    bp_before = jnp.minimum(shifts_before, L - 1)
    widx = sp_after - 1
    code = sp_before | (bp_before << 7) | (widx << 14) | (tr << 21)

    seq2d = sequence.reshape(B * L, H2)
    w_cat = jnp.concatenate([W_ih, W_hh], axis=0)
    u_cat = jnp.concatenate([U_head_w, U_cl, U_cr, W_comp], axis=0)
    bias_g = (b_ih + b_hh).reshape(1, 4 * TD)
    bias_c = U_head_b.reshape(1, 5 * H)

    grid = (B // BB,)
    out = pl.pallas_call(
        functools.partial(_spinn_kernel, T=T, L=L, H=H, TD=TD),
        grid=grid,
        in_specs=[
            pl.BlockSpec((BB, T), lambda i: (i, 0),
                         memory_space=pltpu.SMEM),
            pl.BlockSpec((1, 4 * TD), lambda i: (0, 0)),
            pl.BlockSpec((1, 5 * H), lambda i: (0, 0)),
            pl.BlockSpec((BB, TD), lambda i: (i, 0)),
            pl.BlockSpec((BB, TD), lambda i: (i, 0)),
            pl.BlockSpec(memory_space=pl.ANY),
            pl.BlockSpec(memory_space=pl.ANY),
            pl.BlockSpec(memory_space=pl.ANY),
        ],
        out_specs=pl.BlockSpec((BB, H), lambda i: (i, 0)),
        out_shape=jax.ShapeDtypeStruct((B, H), jnp.float32),
        scratch_shapes=[
            pltpu.VMEM((BB * L, H2), jnp.float32),
            pltpu.VMEM((3 * H + TD, 4 * TD), jnp.float32),
            pltpu.VMEM((3 * H + TD, 5 * H), jnp.float32),
            pltpu.VMEM((BB, H2), jnp.float32),
            pltpu.VMEM((BB, H2), jnp.float32),
            pltpu.VMEM((BB, H2), jnp.float32),
            pltpu.VMEM((BB, H2), jnp.float32),
            pltpu.VMEM((BB, 128), jnp.float32),
            pltpu.VMEM((BB * SS, H2), jnp.float32),
            pltpu.SemaphoreType.DMA,
            pltpu.SemaphoreType.DMA,
            pltpu.SemaphoreType.DMA,
        ],
        compiler_params=pltpu.CompilerParams(
            dimension_semantics=("arbitrary",),
            vmem_limit_bytes=128 * 1024 * 1024,
        ),
        name="spinn_stack_treelstm",
    )(code, bias_g, bias_c, th0, tc0, seq2d, w_cat, u_cat)
    return out


# speedup vs baseline: 1.2115x; 1.2115x over previous
import functools

import jax
import jax.numpy as jnp
from jax.experimental import pallas as pl
from jax.experimental.pallas import tpu as pltpu

BB = 64
SS = 72


def _spinn_kernel(code_ref, bias_g_ref, bias_c_ref, th0_ref, tc0_ref,
                  seq_hbm, w_hbm, u_hbm,
                  out_ref,
                  seq_v, w_v, u_v, top_b, sec_b, buf_b, mval_b,
                  stack_v, sem_seq, sem_w, sem_u,
                  *, T, L, H, TD):
    i = pl.program_id(0)
    nrows = BB * L
    cp_seq = pltpu.make_async_copy(seq_hbm.at[pl.ds(i * nrows, nrows), :],
                                   seq_v, sem_seq)
    cp_w = pltpu.make_async_copy(w_hbm, w_v, sem_w)
    cp_u = pltpu.make_async_copy(u_hbm, u_v, sem_u)
    cp_seq.start()
    cp_w.start()
    cp_u.start()
    cp_seq.wait()

    for mi in range(BB):
        t0 = seq_v[pl.ds(mi * L, 1), :]
        stack_v[pl.ds(mi * SS, 1), :] = t0
        stack_v[pl.ds(mi * SS + 1, 1), :] = t0

    cp_w.wait()
    cp_u.wait()

    def step(t, carry):
        th, tc = carry

        for mi in range(BB):
            c = code_ref[mi, t]
            d = c & 127
            bp = (c >> 7) & 127
            trs = (c >> 21) & 3
            sbase = pl.multiple_of(mi * SS + (((d - 2) >> 3) << 3), 8)
            chunk = stack_v[pl.ds(sbase, 16), :]
            ts = pltpu.roll(chunk, -((d - 2) & 7), axis=0)
            sec_b[pl.ds(mi, 1), :] = ts[0:1, :]
            top_b[pl.ds(mi, 1), :] = ts[1:2, :]
            bbase = pl.multiple_of(mi * L + ((bp >> 3) << 3), 8)
            bchunk = seq_v[pl.ds(bbase, 8), :]
            buf_b[pl.ds(mi, 1), :] = pltpu.roll(bchunk, -(bp & 7), axis=0)[0:1, :]
            mval_b[pl.ds(mi, 1), :] = jnp.full((1, 128), trs, jnp.float32)

        tcol = mval_b[:, 0:1]
        lm = tcol == 2.0
        rm = tcol == 3.0
        sm = tcol == 1.0

        top = top_b[...]
        sec = sec_b[...]
        buf = buf_b[...]
        top_h, top_c = top[:, :H], top[:, H:]
        sec_h, sec_c = sec[:, :H], sec[:, H:]
        buf_h, buf_c = buf[:, :H], buf[:, H:]

        gates = (jnp.dot(buf_h, w_v[0:H, :], preferred_element_type=jnp.float32)
                 + jnp.dot(top_h, w_v[H:2 * H, :], preferred_element_type=jnp.float32)
                 + jnp.dot(sec_h, w_v[2 * H:3 * H, :], preferred_element_type=jnp.float32)
                 + jnp.dot(th, w_v[3 * H:3 * H + TD, :], preferred_element_type=jnp.float32)
                 + bias_g_ref[...])
        ig = gates[:, 0:TD]
        fg = gates[:, TD:2 * TD]
        gg = gates[:, 2 * TD:3 * TD]
        og = gates[:, 3 * TD:4 * TD]
        tc_n = jax.nn.sigmoid(fg) * tc + jax.nn.sigmoid(ig) * jnp.tanh(gg)
        th_n = jax.nn.sigmoid(og) * jnp.tanh(tc_n)

        head_h = jnp.where(lm, top_h, sec_h)
        head_c = jnp.where(lm, top_c, sec_c)
        child_h = jnp.where(lm, sec_h, top_h)
        child_c = jnp.where(lm, sec_c, top_c)
        zeros = jnp.zeros_like(child_h)
        cl_in = jnp.where(lm, child_h, zeros)
        cr_in = jnp.where(rm, child_h, zeros)
        cg = (jnp.dot(head_h.astype(jnp.bfloat16), u_v[0:H, :],
                      preferred_element_type=jnp.float32)
              + jnp.dot(cl_in.astype(jnp.bfloat16), u_v[H:2 * H, :],
                        preferred_element_type=jnp.float32)
              + jnp.dot(cr_in.astype(jnp.bfloat16), u_v[2 * H:3 * H, :],
                        preferred_element_type=jnp.float32)
              + jnp.dot(th_n.astype(jnp.bfloat16), u_v[3 * H:3 * H + TD, :],
                        preferred_element_type=jnp.float32)
              + bias_c_ref[...])
        i_j = cg[:, 0:H]
        o_j = cg[:, H:2 * H]
        f_jh = cg[:, 2 * H:3 * H]
        f_jc = cg[:, 3 * H:4 * H]
        u_j = cg[:, 4 * H:5 * H]
        c_new = (jax.nn.sigmoid(i_j) * jnp.tanh(u_j)
                 + jax.nn.sigmoid(f_jh) * head_c
                 + jax.nn.sigmoid(f_jc) * child_c)
        h_new = jax.nn.sigmoid(o_j) * jnp.tanh(c_new)

        push_h = jnp.where(sm, buf_h, h_new)
        push_c = jnp.where(sm, buf_c, c_new)
        push = jnp.concatenate([push_h, push_c], axis=1)

        iota8 = jax.lax.broadcasted_iota(jnp.int32, (8, 1), 0)
        for mi in range(BB):
            c = code_ref[mi, t]
            w = (c >> 14) & 127
            wbase = pl.multiple_of(mi * SS + ((w >> 3) << 3), 8)
            ch = stack_v[pl.ds(wbase, 8), :]
            pb = jnp.broadcast_to(push[mi:mi + 1, :], (8, 2 * H))
            stack_v[pl.ds(wbase, 8), :] = jnp.where(iota8 == (w & 7), pb, ch)

        return th_n, tc_n

    jax.lax.fori_loop(0, T, step, (th0_ref[...], tc0_ref[...]))

    for mi in range(BB):
        c = code_ref[mi, T - 1]
        w = (c >> 14) & 127
        fbase = pl.multiple_of(mi * SS + ((w >> 3) << 3), 8)
        fch = stack_v[pl.ds(fbase, 8), :]
        out_ref[pl.ds(mi, 1), :] = pltpu.roll(fch, -(w & 7), axis=0)[0:1, 0:H]


@jax.jit
def kernel(sequence, transitions, W_comp, U_head_w, U_head_b, U_cl, U_cr,
           W_ih, W_hh, b_ih, b_hh, th0, tc0):
    B, L, H2 = sequence.shape
    H = H2 // 2
    TD = W_hh.shape[0]
    T = transitions.shape[1]

    tr = transitions.astype(jnp.int32)
    is_sh = (tr == 1).astype(jnp.int32)
    sp_after = 2 + jnp.cumsum(2 * is_sh - 1, axis=1)
    sp_before = jnp.concatenate(
        [jnp.full((B, 1), 2, jnp.int32), sp_after[:, :-1]], axis=1)
    shifts_before = jnp.concatenate(
        [jnp.zeros((B, 1), jnp.int32), jnp.cumsum(is_sh, axis=1)[:, :-1]], axis=1)
    bp_before = jnp.minimum(shifts_before, L - 1)
    widx = sp_after - 1
    code = sp_before | (bp_before << 7) | (widx << 14) | (tr << 21)

    seq2d = sequence.reshape(B * L, H2)
    w_cat = jnp.concatenate([W_ih, W_hh], axis=0)
    u_cat = jnp.concatenate([U_head_w, U_cl, U_cr, W_comp],
                            axis=0).astype(jnp.bfloat16)
    bias_g = (b_ih + b_hh).reshape(1, 4 * TD)
    bias_c = U_head_b.reshape(1, 5 * H)

    grid = (B // BB,)
    out = pl.pallas_call(
        functools.partial(_spinn_kernel, T=T, L=L, H=H, TD=TD),
        grid=grid,
        in_specs=[
            pl.BlockSpec((BB, T), lambda i: (i, 0),
                         memory_space=pltpu.SMEM),
            pl.BlockSpec((1, 4 * TD), lambda i: (0, 0)),
            pl.BlockSpec((1, 5 * H), lambda i: (0, 0)),
            pl.BlockSpec((BB, TD), lambda i: (i, 0)),
            pl.BlockSpec((BB, TD), lambda i: (i, 0)),
            pl.BlockSpec(memory_space=pl.ANY),
            pl.BlockSpec(memory_space=pl.ANY),
            pl.BlockSpec(memory_space=pl.ANY),
        ],
        out_specs=pl.BlockSpec((BB, H), lambda i: (i, 0)),
        out_shape=jax.ShapeDtypeStruct((B, H), jnp.float32),
        scratch_shapes=[
            pltpu.VMEM((BB * L, H2), jnp.float32),
            pltpu.VMEM((3 * H + TD, 4 * TD), jnp.float32),
            pltpu.VMEM((3 * H + TD, 5 * H), jnp.bfloat16),
            pltpu.VMEM((BB, H2), jnp.float32),
            pltpu.VMEM((BB, H2), jnp.float32),
            pltpu.VMEM((BB, H2), jnp.float32),
            pltpu.VMEM((BB, 128), jnp.float32),
            pltpu.VMEM((BB * SS, H2), jnp.float32),
            pltpu.SemaphoreType.DMA,
            pltpu.SemaphoreType.DMA,
            pltpu.SemaphoreType.DMA,
        ],
        compiler_params=pltpu.CompilerParams(
            dimension_semantics=("arbitrary",),
            vmem_limit_bytes=128 * 1024 * 1024,
        ),
        name="spinn_stack_treelstm",
    )(code, bias_g, bias_c, th0, tc0, seq2d, w_cat, u_cat)
    return out


# speedup vs baseline: 1.2448x; 1.0274x over previous
import functools

import jax
import jax.numpy as jnp
from jax.experimental import pallas as pl
from jax.experimental.pallas import tpu as pltpu

BB = 64
SS = 72


def _spinn_kernel(code_ref, bias_g_ref, bias_c_ref, th0_ref, tc0_ref,
                  seq_hbm, w_hbm, u_hbm,
                  out_ref,
                  seq_v, w_v, u_v, top_b, sec_b, buf_b, mval_b,
                  stack_v, sem_seq, sem_w, sem_u,
                  *, T, L, H, TD):
    i = pl.program_id(0)
    nrows = BB * L
    cp_seq = pltpu.make_async_copy(seq_hbm.at[pl.ds(i * nrows, nrows), :],
                                   seq_v, sem_seq)
    cp_w = pltpu.make_async_copy(w_hbm, w_v, sem_w)
    cp_u = pltpu.make_async_copy(u_hbm, u_v, sem_u)
    cp_seq.start()
    cp_w.start()
    cp_u.start()
    cp_seq.wait()

    for mi in range(BB):
        t0 = seq_v[pl.ds(mi * L, 1), :]
        stack_v[pl.ds(mi * SS, 1), :] = t0
        stack_v[pl.ds(mi * SS + 1, 1), :] = t0

    cp_w.wait()
    cp_u.wait()

    def step(t, carry):
        th, tc = carry

        for mi in range(BB):
            c = code_ref[mi, t]
            d = c & 127
            bp = (c >> 7) & 127
            trs = (c >> 21) & 3
            sbase = pl.multiple_of(mi * SS + (((d - 2) >> 3) << 3), 8)
            chunk = stack_v[pl.ds(sbase, 16), :]
            ts = pltpu.roll(chunk, -((d - 2) & 7), axis=0)
            sec_b[pl.ds(mi, 1), :] = ts[0:1, :]
            top_b[pl.ds(mi, 1), :] = ts[1:2, :]
            bbase = pl.multiple_of(mi * L + ((bp >> 3) << 3), 8)
            bchunk = seq_v[pl.ds(bbase, 8), :]
            buf_b[pl.ds(mi, 1), :] = pltpu.roll(bchunk, -(bp & 7), axis=0)[0:1, :]
            mval_b[pl.ds(mi, 1), :] = jnp.full((1, 128), trs, jnp.float32)

        tcol = mval_b[:, 0:1]
        lm = tcol == 2.0
        rm = tcol == 3.0
        sm = tcol == 1.0

        top = top_b[...]
        sec = sec_b[...]
        buf = buf_b[...]
        top_h, top_c = top[:, :H], top[:, H:]
        sec_h, sec_c = sec[:, :H], sec[:, H:]
        buf_h, buf_c = buf[:, :H], buf[:, H:]

        gates = (jnp.dot(buf_h, w_v[0:H, :], preferred_element_type=jnp.float32)
                 + jnp.dot(top_h, w_v[H:2 * H, :], preferred_element_type=jnp.float32)
                 + jnp.dot(sec_h, w_v[2 * H:3 * H, :], preferred_element_type=jnp.float32)
                 + jnp.dot(th, w_v[3 * H:3 * H + TD, :], preferred_element_type=jnp.float32)
                 + bias_g_ref[...])
        ig = gates[:, 0:TD]
        fg = gates[:, TD:2 * TD]
        gg = gates[:, 2 * TD:3 * TD]
        og = gates[:, 3 * TD:4 * TD]
        tc_n = jax.nn.sigmoid(fg) * tc + jax.nn.sigmoid(ig) * jnp.tanh(gg)
        th_n = jax.nn.sigmoid(og) * jnp.tanh(tc_n)

        head_h = jnp.where(lm, top_h, sec_h)
        head_c = jnp.where(lm, top_c, sec_c)
        child_h = jnp.where(lm, sec_h, top_h)
        child_c = jnp.where(lm, sec_c, top_c)
        zeros = jnp.zeros_like(child_h)
        cl_in = jnp.where(lm, child_h, zeros)
        cr_in = jnp.where(rm, child_h, zeros)
        cg = (jnp.dot(head_h, u_v[0:H, :], preferred_element_type=jnp.float32)
              + jnp.dot(cl_in, u_v[H:2 * H, :], preferred_element_type=jnp.float32)
              + jnp.dot(cr_in, u_v[2 * H:3 * H, :], preferred_element_type=jnp.float32)
              + jnp.dot(th_n, u_v[3 * H:3 * H + TD, :], preferred_element_type=jnp.float32)
              + bias_c_ref[...])
        i_j = cg[:, 0:H]
        o_j = cg[:, H:2 * H]
        f_jh = cg[:, 2 * H:3 * H]
        f_jc = cg[:, 3 * H:4 * H]
        u_j = cg[:, 4 * H:5 * H]
        c_new = (jax.nn.sigmoid(i_j) * jnp.tanh(u_j)
                 + jax.nn.sigmoid(f_jh) * head_c
                 + jax.nn.sigmoid(f_jc) * child_c)
        h_new = jax.nn.sigmoid(o_j) * jnp.tanh(c_new)

        push_h = jnp.where(sm, buf_h, h_new)
        push_c = jnp.where(sm, buf_c, c_new)
        push = jnp.concatenate([push_h, push_c], axis=1)

        iota8 = jax.lax.broadcasted_iota(jnp.int32, (8, 1), 0)
        for mi in range(BB):
            c = code_ref[mi, t]
            w = (c >> 14) & 127
            wbase = pl.multiple_of(mi * SS + ((w >> 3) << 3), 8)
            ch = stack_v[pl.ds(wbase, 8), :]
            pb = jnp.broadcast_to(push[mi:mi + 1, :], (8, 2 * H))
            stack_v[pl.ds(wbase, 8), :] = jnp.where(iota8 == (w & 7), pb, ch)

        return th_n, tc_n

    jax.lax.fori_loop(0, T, step, (th0_ref[...], tc0_ref[...]))

    for mi in range(BB):
        c = code_ref[mi, T - 1]
        w = (c >> 14) & 127
        fbase = pl.multiple_of(mi * SS + ((w >> 3) << 3), 8)
        fch = stack_v[pl.ds(fbase, 8), :]
        out_ref[pl.ds(mi, 1), :] = pltpu.roll(fch, -(w & 7), axis=0)[0:1, 0:H]


@jax.jit
def kernel(sequence, transitions, W_comp, U_head_w, U_head_b, U_cl, U_cr,
           W_ih, W_hh, b_ih, b_hh, th0, tc0):
    B, L, H2 = sequence.shape
    H = H2 // 2
    TD = W_hh.shape[0]
    T = transitions.shape[1]

    tr = transitions.astype(jnp.int32)
    is_sh = (tr == 1).astype(jnp.int32)
    sp_after = 2 + jnp.cumsum(2 * is_sh - 1, axis=1)
    sp_before = jnp.concatenate(
        [jnp.full((B, 1), 2, jnp.int32), sp_after[:, :-1]], axis=1)
    shifts_before = jnp.concatenate(
        [jnp.zeros((B, 1), jnp.int32), jnp.cumsum(is_sh, axis=1)[:, :-1]], axis=1)
    bp_before = jnp.minimum(shifts_before, L - 1)
    widx = sp_after - 1
    code = sp_before | (bp_before << 7) | (widx << 14) | (tr << 21)

    seq2d = sequence.reshape(B * L, H2)
    w_cat = jnp.concatenate([W_ih, W_hh], axis=0)
    u_cat = jnp.concatenate([U_head_w, U_cl, U_cr, W_comp], axis=0)
    bias_g = (b_ih + b_hh).reshape(1, 4 * TD)
    bias_c = U_head_b.reshape(1, 5 * H)

    grid = (B // BB,)
    out = pl.pallas_call(
        functools.partial(_spinn_kernel, T=T, L=L, H=H, TD=TD),
        grid=grid,
        in_specs=[
            pl.BlockSpec((BB, T), lambda i: (i, 0),
                         memory_space=pltpu.SMEM),
            pl.BlockSpec((1, 4 * TD), lambda i: (0, 0)),
            pl.BlockSpec((1, 5 * H), lambda i: (0, 0)),
            pl.BlockSpec((BB, TD), lambda i: (i, 0)),
            pl.BlockSpec((BB, TD), lambda i: (i, 0)),
            pl.BlockSpec(memory_space=pl.ANY),
            pl.BlockSpec(memory_space=pl.ANY),
            pl.BlockSpec(memory_space=pl.ANY),
        ],
        out_specs=pl.BlockSpec((BB, H), lambda i: (i, 0)),
        out_shape=jax.ShapeDtypeStruct((B, H), jnp.float32),
        scratch_shapes=[
            pltpu.VMEM((BB * L, H2), jnp.float32),
            pltpu.VMEM((3 * H + TD, 4 * TD), jnp.float32),
            pltpu.VMEM((3 * H + TD, 5 * H), jnp.float32),
            pltpu.VMEM((BB, H2), jnp.float32),
            pltpu.VMEM((BB, H2), jnp.float32),
            pltpu.VMEM((BB, H2), jnp.float32),
            pltpu.VMEM((BB, 128), jnp.float32),
            pltpu.VMEM((BB * SS, H2), jnp.float32),
            pltpu.SemaphoreType.DMA,
            pltpu.SemaphoreType.DMA,
            pltpu.SemaphoreType.DMA,
        ],
        compiler_params=pltpu.CompilerParams(
            dimension_semantics=("arbitrary",),
            vmem_limit_bytes=128 * 1024 * 1024,
        ),
        name="spinn_stack_treelstm",
    )(code, bias_g, bias_c, th0, tc0, seq2d, w_cat, u_cat)
    return out
